# baseline (speedup 1.0000x reference)
import functools

import jax
import jax.numpy as jnp
from jax import lax
from jax.experimental import pallas as pl
from jax.experimental.pallas import tpu as pltpu
from jax.experimental.pallas import tpu_sc as plsc

N = 10000
E = 320000
D = 128

NC = 2
NS = 16
NW = NC * NS
EPW = E // NW
CHUNK = 80
NCHUNK = EPW // CHUNK
NPAD = 10240
RPT = NPAD // NS
ZROWS = 32

_mesh = plsc.VectorSubcoreMesh(core_axis_name="c", subcore_axis_name="s")


def _deg_body(src_hbm, dst_hbm, out_hbm,
              degs_v, degd_v, idxb, tmpv, accv, shr, sem):
    cid = lax.axis_index("c")
    sid = lax.axis_index("s")
    wid = cid * NS + sid

    zrow = jnp.zeros((16,), jnp.float32)
    onerow = jnp.ones((16,), jnp.float32)

    def fill_z(i, _):
        degs_v[pl.ds(i * 16, 16)] = zrow
        degd_v[pl.ds(i * 16, 16)] = zrow
        return 0
    lax.fori_loop(0, NPAD // 16, fill_z, 0)

    ebase = wid * EPW
    pltpu.sync_copy(src_hbm.at[pl.ds(ebase, EPW)], idxb)

    def acc_s(g, _):
        ix = idxb[pl.ds(g * 16, 16)]
        plsc.addupdate_scatter(degs_v, [ix], onerow)
        return 0
    lax.fori_loop(0, EPW // 16, acc_s, 0)

    pltpu.sync_copy(dst_hbm.at[pl.ds(ebase, EPW)], idxb)

    def acc_d(g, _):
        ix = idxb[pl.ds(g * 16, 16)]
        plsc.addupdate_scatter(degd_v, [ix], onerow)
        return 0
    lax.fori_loop(0, EPW // 16, acc_d, 0)

    pltpu.sync_copy(degs_v, shr.at[pl.ds(sid * NPAD, NPAD)])
    pltpu.sync_copy(degd_v, shr.at[pl.ds((NS + sid) * NPAD, NPAD)])
    plsc.subcore_barrier()

    r0 = sid * RPT
    for which in range(2):
        def fill_za(i, _):
            accv[pl.ds(i * 16, 16)] = zrow
            return 0
        lax.fori_loop(0, RPT // 16, fill_za, 0)
        for k in range(NS):
            pltpu.sync_copy(
                shr.at[pl.ds((which * NS + k) * NPAD + r0, RPT)], tmpv)

            def red(i, _):
                accv[pl.ds(i * 16, 16)] = (accv[pl.ds(i * 16, 16)]
                                           + tmpv[pl.ds(i * 16, 16)])
                return 0
            lax.fori_loop(0, RPT // 16, red, 0)
        pltpu.sync_copy(
            accv, out_hbm.at[pl.ds((cid * 2 + which) * NPAD + r0, RPT)])


_deg_call = pl.kernel(
    _deg_body,
    out_type=jax.ShapeDtypeStruct((NC * 2 * NPAD,), jnp.float32),
    mesh=_mesh,
    compiler_params=pltpu.CompilerParams(needs_layout_passes=False),
    scratch_types=[
        pltpu.VMEM((NPAD,), jnp.float32),
        pltpu.VMEM((NPAD,), jnp.float32),
        pltpu.VMEM((EPW,), jnp.int32),
        pltpu.VMEM((RPT,), jnp.float32),
        pltpu.VMEM((RPT,), jnp.float32),
        pltpu.VMEM_SHARED((2 * NS * NPAD,), jnp.float32),
        pltpu.SemaphoreType.DMA,
    ],
)


def _agg_body(hp_hbm, src_hbm, dst3_hbm, mask_hbm, out_hbm,
              hagg, rows2, scaled, srcb, dstb, maskd, gsem, msem):
    cid = lax.axis_index("c")
    sid = lax.axis_index("s")
    wid = cid * NS + sid

    zrow = jnp.zeros((16,), jnp.float32)
    ebase = wid * EPW

    pltpu.sync_copy(src_hbm.at[pl.ds(ebase, EPW)], srcb)
    pltpu.sync_copy(dst3_hbm.at[wid], dstb)
    pltpu.sync_copy(mask_hbm.at[pl.ds(ebase, CHUNK)], maskd.at[0])
    pltpu.async_copy(hp_hbm.at[srcb.at[pl.ds(0, CHUNK)]], rows2.at[0], gsem)

    def fill_z(i, _):
        for j in range(D // 16):
            scaled[i, pl.ds(j * 16, 16)] = zrow
        return 0
    lax.fori_loop(0, CHUNK, fill_z, 0)

    r0 = sid * RPT
    for k in range(RPT // CHUNK):
        pltpu.sync_copy(scaled, hagg.at[pl.ds(r0 + k * CHUNK, CHUNK)])
    plsc.subcore_barrier()

    pltpu.async_copy(hp_hbm.at[srcb.at[pl.ds(CHUNK, CHUNK)]],
                     rows2.at[1], gsem)
    pltpu.async_copy(mask_hbm.at[pl.ds(ebase + CHUNK, CHUNK)],
                     maskd.at[1], msem)

    def _process(i, b):
        pltpu.make_async_copy(
            hp_hbm.at[srcb.at[pl.ds(i * CHUNK, CHUNK)]], rows2.at[b],
            gsem).wait()

        @pl.when(i > 0)
        def _():
            pltpu.make_async_copy(
                mask_hbm.at[pl.ds(ebase + i * CHUNK, CHUNK)], maskd.at[b],
                msem).wait()

        def grp(g, _):
            mv = maskd[b, pl.ds(g * 16, 16)]
            e0 = g * 16
            for l in range(16):
                m = mv[l]
                e = e0 + l
                for j in range(D // 32):
                    v = rows2[b, e, pl.ds(j * 16, 16)]
                    lo, hi = plsc.unpack(
                        plsc.bitcast(v, jnp.bfloat16),
                        format=plsc.PackFormat.INTERLEAVED)
                    scaled[e, pl.ds(j * 32, 16)] = lo * m
                    scaled[e, pl.ds(j * 32 + 16, 16)] = hi * m
            return 0
        lax.fori_loop(0, CHUNK // 16, grp, 0)
        pltpu.sync_copy(scaled, hagg.at[dstb.at[i]], add=True)

        @pl.when(i + 2 < NCHUNK)
        def _():
            pltpu.async_copy(
                hp_hbm.at[srcb.at[pl.ds((i + 2) * CHUNK, CHUNK)]],
                rows2.at[b], gsem)
            pltpu.async_copy(
                mask_hbm.at[pl.ds(ebase + (i + 2) * CHUNK, CHUNK)],
                maskd.at[b], msem)

    def pair(p, _):
        _process(2 * p, 0)
        _process(2 * p + 1, 1)
        return 0
    lax.fori_loop(0, NCHUNK // 2, pair, 0)
    _process(jnp.int32(NCHUNK - 1), 0)
    plsc.subcore_barrier()

    pltpu.sync_copy(hagg.at[pl.ds(r0, RPT)], out_hbm.at[cid, pl.ds(r0, RPT)])


_agg_call = pl.kernel(
    _agg_body,
    out_type=jax.ShapeDtypeStruct((NC, NPAD, D), jnp.float32),
    mesh=_mesh,
    compiler_params=pltpu.CompilerParams(
        needs_layout_passes=False, use_tc_tiling_on_sc=False),
    scratch_types=[
        pltpu.VMEM_SHARED((NPAD, D), jnp.float32),
        pltpu.VMEM((2, CHUNK, D // 2), jnp.float32),
        pltpu.VMEM((CHUNK, D), jnp.float32),
        pltpu.VMEM((EPW,), jnp.int32),
        pltpu.VMEM((NCHUNK, CHUNK), jnp.int32),
        pltpu.VMEM((2, CHUNK), jnp.float32),
        pltpu.SemaphoreType.DMA,
        pltpu.SemaphoreType.DMA,
    ],
)

BN = 1000


def _norm_body(s0_ref, s1_ref, h_ref, out_ref):
    deg = s0_ref[...] + s1_ref[...]
    norm = lax.rsqrt(jnp.maximum(deg, 1.0))
    y = (h_ref[...] * norm).astype(jnp.bfloat16)
    out_ref[...] = y.reshape(BN, 4, 2, 16).transpose(0, 1, 3, 2).reshape(BN, D)


def _norm_call(s0, s1, h):
    return pl.pallas_call(
        _norm_body,
        grid=(N // BN,),
        in_specs=[
            pl.BlockSpec((BN, 1), lambda i: (i, 0)),
            pl.BlockSpec((BN, 1), lambda i: (i, 0)),
            pl.BlockSpec((BN, D), lambda i: (i, 0)),
        ],
        out_specs=pl.BlockSpec((BN, D), lambda i: (i, 0)),
        out_shape=jax.ShapeDtypeStruct((N, D), jnp.bfloat16),
    )(s0, s1, h)


def _final_body(hp_ref, h_ref, s0_ref, s1_ref, d0_ref, d1_ref, w_ref,
                b_ref, out_ref):
    hagg = hp_ref[0] + hp_ref[1]
    h2 = jnp.dot(hagg, w_ref[...], preferred_element_type=jnp.float32) + b_ref[...]
    deg = d0_ref[...] + d1_ref[...]
    innorm = lax.rsqrt(jnp.maximum(deg, 1.0))
    odeg = s0_ref[...] + s1_ref[...]
    onorm = lax.rsqrt(jnp.maximum(odeg, 1.0))
    out_ref[...] = h_ref[...] * onorm + jnp.maximum(h2 * innorm, 0.0)


def _final_call(hpart, h, s0, s1, d0, d1, W, b2):
    return pl.pallas_call(
        _final_body,
        grid=(N // BN,),
        in_specs=[
            pl.BlockSpec((NC, BN, D), lambda i: (0, i, 0)),
            pl.BlockSpec((BN, D), lambda i: (i, 0)),
            pl.BlockSpec((BN, 1), lambda i: (i, 0)),
            pl.BlockSpec((BN, 1), lambda i: (i, 0)),
            pl.BlockSpec((BN, 1), lambda i: (i, 0)),
            pl.BlockSpec((BN, 1), lambda i: (i, 0)),
            pl.BlockSpec((D, D), lambda i: (0, 0)),
            pl.BlockSpec((1, D), lambda i: (0, 0)),
        ],
        out_specs=pl.BlockSpec((BN, D), lambda i: (i, 0)),
        out_shape=jax.ShapeDtypeStruct((N, D), jnp.float32),
    )(hpart, h, s0, s1, d0, d1, W, b2)


def kernel(h, edge_index, edge_mask, W, b):
    src = edge_index[0]
    dst = edge_index[1]
    dst3 = dst.reshape(NW, NCHUNK, CHUNK)
    mask1 = edge_mask.reshape(E)
    deg4 = _deg_call(src, dst).reshape(NC * 2, NPAD)
    s0 = deg4[0, :N].reshape(N, 1)
    d0 = deg4[1, :N].reshape(N, 1)
    s1 = deg4[2, :N].reshape(N, 1)
    d1 = deg4[3, :N].reshape(N, 1)
    h_perm = _norm_call(s0, s1, h)
    h_packed = lax.bitcast_convert_type(h_perm.reshape(N, D // 2, 2),
                                        jnp.float32)
    hpart = _agg_call(h_packed, src, dst3, mask1)
    return _final_call(hpart, h, s0, s1, d0, d1, W, b.reshape(1, D))


# speedup vs baseline: 2.2422x; 2.2422x over previous
import jax
import jax.numpy as jnp
from jax import lax
from jax.experimental import pallas as pl
from jax.experimental.pallas import tpu as pltpu
from jax.experimental.pallas import tpu_sc as plsc

N = 10000
E = 320000
D = 128

NC = 2
NS = 16
NW = NC * NS
EPW = E // NW
CHUNK = 80
NCHUNK = EPW // CHUNK
NPAD = 10240
RPT = NPAD // NS

_mesh = plsc.VectorSubcoreMesh(core_axis_name="c", subcore_axis_name="s")


def _deg_body(src_hbm, dst_hbm, out_hbm,
              degs_v, degd_v, idxb, tmpv, accv, shr, sem):
    cid = lax.axis_index("c")
    sid = lax.axis_index("s")
    wid = cid * NS + sid

    zrow = jnp.zeros((16,), jnp.float32)
    onerow = jnp.ones((16,), jnp.float32)

    def fill_z(i, _):
        degs_v[pl.ds(i * 16, 16)] = zrow
        degd_v[pl.ds(i * 16, 16)] = zrow
        return 0
    lax.fori_loop(0, NPAD // 16, fill_z, 0)

    ebase = wid * EPW
    pltpu.sync_copy(src_hbm.at[pl.ds(ebase, EPW)], idxb)

    def acc_s(g, _):
        ix = idxb[pl.ds(g * 16, 16)]
        plsc.addupdate_scatter(degs_v, [ix], onerow)
        return 0
    lax.fori_loop(0, EPW // 16, acc_s, 0)

    pltpu.sync_copy(dst_hbm.at[pl.ds(ebase, EPW)], idxb)

    def acc_d(g, _):
        ix = idxb[pl.ds(g * 16, 16)]
        plsc.addupdate_scatter(degd_v, [ix], onerow)
        return 0
    lax.fori_loop(0, EPW // 16, acc_d, 0)

    pltpu.sync_copy(degs_v, shr.at[pl.ds(sid * NPAD, NPAD)])
    pltpu.sync_copy(degd_v, shr.at[pl.ds((NS + sid) * NPAD, NPAD)])
    plsc.subcore_barrier()

    r0 = sid * RPT
    for which in range(2):
        def fill_za(i, _):
            accv[pl.ds(i * 16, 16)] = zrow
            return 0
        lax.fori_loop(0, RPT // 16, fill_za, 0)
        for k in range(NS):
            pltpu.sync_copy(
                shr.at[pl.ds((which * NS + k) * NPAD + r0, RPT)], tmpv)

            def red(i, _):
                accv[pl.ds(i * 16, 16)] = (accv[pl.ds(i * 16, 16)]
                                           + tmpv[pl.ds(i * 16, 16)])
                return 0
            lax.fori_loop(0, RPT // 16, red, 0)
        pltpu.sync_copy(
            accv, out_hbm.at[pl.ds((cid * 2 + which) * NPAD + r0, RPT)])


_deg_call = pl.kernel(
    _deg_body,
    out_type=jax.ShapeDtypeStruct((NC * 2 * NPAD,), jnp.float32),
    mesh=_mesh,
    compiler_params=pltpu.CompilerParams(needs_layout_passes=False),
    scratch_types=[
        pltpu.VMEM((NPAD,), jnp.float32),
        pltpu.VMEM((NPAD,), jnp.float32),
        pltpu.VMEM((EPW,), jnp.int32),
        pltpu.VMEM((RPT,), jnp.float32),
        pltpu.VMEM((RPT,), jnp.float32),
        pltpu.VMEM_SHARED((2 * NS * NPAD,), jnp.float32),
        pltpu.SemaphoreType.DMA,
    ],
)


def _agg_body(hn_hbm, src_hbm, dst3_hbm, mask_hbm, out_hbm,
              hagg, rows2, srcb, dstb, maskd, gsem, msem):
    cid = lax.axis_index("c")
    sid = lax.axis_index("s")
    wid = cid * NS + sid

    zrow = jnp.zeros((16,), jnp.float32)
    ebase = wid * EPW

    pltpu.sync_copy(src_hbm.at[pl.ds(ebase, EPW)], srcb)
    pltpu.sync_copy(dst3_hbm.at[wid], dstb)
    pltpu.sync_copy(mask_hbm.at[pl.ds(ebase, CHUNK)], maskd.at[0])
    pltpu.async_copy(hn_hbm.at[srcb.at[pl.ds(0, CHUNK)]], rows2.at[0], gsem)

    def fill_z(i, _):
        for j in range(D // 16):
            rows2[1, i, pl.ds(j * 16, 16)] = zrow
        return 0
    lax.fori_loop(0, CHUNK, fill_z, 0)

    r0 = sid * RPT
    for k in range(RPT // CHUNK):
        pltpu.sync_copy(rows2.at[1], hagg.at[pl.ds(r0 + k * CHUNK, CHUNK)])
    plsc.subcore_barrier()

    pltpu.async_copy(hn_hbm.at[srcb.at[pl.ds(CHUNK, CHUNK)]],
                     rows2.at[1], gsem)
    pltpu.async_copy(mask_hbm.at[pl.ds(ebase + CHUNK, CHUNK)],
                     maskd.at[1], msem)

    def _process(i, b):
        pltpu.make_async_copy(
            hn_hbm.at[srcb.at[pl.ds(i * CHUNK, CHUNK)]], rows2.at[b],
            gsem).wait()

        @pl.when(i > 0)
        def _():
            pltpu.make_async_copy(
                mask_hbm.at[pl.ds(ebase + i * CHUNK, CHUNK)], maskd.at[b],
                msem).wait()

        def grp(g, _):
            mv = maskd[b, pl.ds(g * 16, 16)]
            e0 = g * 16
            for l in range(16):
                m = mv[l]
                e = e0 + l
                for j in range(D // 16):
                    rows2[b, e, pl.ds(j * 16, 16)] = (
                        rows2[b, e, pl.ds(j * 16, 16)] * m)
            return 0
        lax.fori_loop(0, CHUNK // 16, grp, 0)
        pltpu.sync_copy(rows2.at[b], hagg.at[dstb.at[i]], add=True)

        @pl.when(i + 2 < NCHUNK)
        def _():
            pltpu.async_copy(
                hn_hbm.at[srcb.at[pl.ds((i + 2) * CHUNK, CHUNK)]],
                rows2.at[b], gsem)
            pltpu.async_copy(
                mask_hbm.at[pl.ds(ebase + (i + 2) * CHUNK, CHUNK)],
                maskd.at[b], msem)

    def pair(p, _):
        _process(2 * p, 0)
        _process(2 * p + 1, 1)
        return 0
    lax.fori_loop(0, NCHUNK // 2, pair, 0)
    _process(jnp.int32(NCHUNK - 1), 0)
    plsc.subcore_barrier()

    pltpu.sync_copy(hagg.at[pl.ds(r0, RPT)], out_hbm.at[cid, pl.ds(r0, RPT)])


_agg_call = pl.kernel(
    _agg_body,
    out_type=jax.ShapeDtypeStruct((NC, NPAD, D), jnp.float32),
    mesh=_mesh,
    scratch_types=[
        pltpu.VMEM_SHARED((NPAD, D), jnp.float32),
        pltpu.VMEM((2, CHUNK, D), jnp.float32),
        pltpu.VMEM((EPW,), jnp.int32),
        pltpu.VMEM((NCHUNK, CHUNK), jnp.int32),
        pltpu.VMEM((2, CHUNK), jnp.float32),
        pltpu.SemaphoreType.DMA,
        pltpu.SemaphoreType.DMA,
    ],
)

BN = 1000


def _norm_body(s0_ref, s1_ref, h_ref, out_ref):
    deg = s0_ref[...] + s1_ref[...]
    norm = lax.rsqrt(jnp.maximum(deg, 1.0))
    out_ref[...] = h_ref[...] * norm


def _norm_call(s0, s1, h):
    return pl.pallas_call(
        _norm_body,
        grid=(N // BN,),
        in_specs=[
            pl.BlockSpec((BN, 1), lambda i: (i, 0)),
            pl.BlockSpec((BN, 1), lambda i: (i, 0)),
            pl.BlockSpec((BN, D), lambda i: (i, 0)),
        ],
        out_specs=pl.BlockSpec((BN, D), lambda i: (i, 0)),
        out_shape=jax.ShapeDtypeStruct((N, D), jnp.float32),
    )(s0, s1, h)


def _final_body(hp_ref, hn_ref, d0_ref, d1_ref, w_ref, b_ref, out_ref):
    hagg = hp_ref[0] + hp_ref[1]
    h2 = jnp.dot(hagg, w_ref[...], preferred_element_type=jnp.float32) + b_ref[...]
    deg = d0_ref[...] + d1_ref[...]
    innorm = lax.rsqrt(jnp.maximum(deg, 1.0))
    out_ref[...] = hn_ref[...] + jnp.maximum(h2 * innorm, 0.0)


def _final_call(hpart, h_norm, d0, d1, W, b2):
    return pl.pallas_call(
        _final_body,
        grid=(N // BN,),
        in_specs=[
            pl.BlockSpec((NC, BN, D), lambda i: (0, i, 0)),
            pl.BlockSpec((BN, D), lambda i: (i, 0)),
            pl.BlockSpec((BN, 1), lambda i: (i, 0)),
            pl.BlockSpec((BN, 1), lambda i: (i, 0)),
            pl.BlockSpec((D, D), lambda i: (0, 0)),
            pl.BlockSpec((1, D), lambda i: (0, 0)),
        ],
        out_specs=pl.BlockSpec((BN, D), lambda i: (i, 0)),
        out_shape=jax.ShapeDtypeStruct((N, D), jnp.float32),
    )(hpart, h_norm, d0, d1, W, b2)


def kernel(h, edge_index, edge_mask, W, b):
    src = edge_index[0]
    dst = edge_index[1]
    dst3 = dst.reshape(NW, NCHUNK, CHUNK)
    mask1 = edge_mask.reshape(E)
    deg4 = _deg_call(src, dst).reshape(NC * 2, NPAD)
    s0 = deg4[0, :N].reshape(N, 1)
    d0 = deg4[1, :N].reshape(N, 1)
    s1 = deg4[2, :N].reshape(N, 1)
    d1 = deg4[3, :N].reshape(N, 1)
    h_norm = _norm_call(s0, s1, h)
    hpart = _agg_call(h_norm, src, dst3, mask1)
    return _final_call(hpart, h_norm, d0, d1, W, b.reshape(1, D))
